# baseline (speedup 1.0000x reference)
import functools

import jax
import jax.numpy as jnp
from jax import lax
from jax.experimental import pallas as pl
from jax.experimental.pallas import tpu as pltpu
from jax.experimental.pallas import tpu_sc as plsc

N = 320000
D = 128
G = 128
VIRT = 100
NC = 2
NS = 16
NW = NC * NS
NV = D // 16
TG = 2 * G

NSC = 128000
RW = NSC // NW
C = 80
NCH = RW // C
NBUF = 5
PF = 4
NQ = C // 16

RTC = 1280
NBT = (N - NSC) // RTC
TC0 = NSC // RTC


@functools.partial(
    pl.kernel,
    mesh=plsc.VectorSubcoreMesh(core_axis_name="c", subcore_axis_name="s"),
    out_type=jax.ShapeDtypeStruct((NC * TG * D,), jnp.float32),
    scratch_types=(
        [pltpu.VMEM((RW + 16,), jnp.int32),
         pltpu.VMEM((RW + 16,), jnp.int32)]
        + [pltpu.VMEM((C, D), jnp.float32)] * NBUF
        + [pltpu.VMEM((2 * D,), jnp.float32),
           pltpu.VMEM((16 * D,), jnp.float32),
           pltpu.VMEM((16, D), jnp.float32),
           pltpu.VMEM((16 * D,), jnp.float32),
           pltpu.VMEM((16 * D,), jnp.float32),
           pltpu.VMEM((16, D), jnp.float32),
           pltpu.VMEM((C,), jnp.int32),
           pltpu.VMEM_SHARED((NS * TG * D,), jnp.float32),
           pltpu.VMEM_SHARED((TG, D), jnp.float32)]
        + [pltpu.SemaphoreType.DMA] * NBUF
    ),
)
def _pool_kernel(x_hbm, z_hbm, b_hbm, out_hbm, z_v, b_v, *refs):
    rows = refs[0:NBUF]
    stage = refs[NBUF]
    zbuf = refs[NBUF + 1]
    zbuf2 = refs[NBUF + 2]
    red = refs[NBUF + 3]
    tmp = refs[NBUF + 4]
    tmp2 = refs[NBUF + 5]
    dsts = refs[NBUF + 6]
    slices = refs[NBUF + 7]
    acc_sc = refs[NBUF + 8]
    sem_row = refs[NBUF + 9:NBUF + 9 + NBUF]

    cid = lax.axis_index("c")
    sid = lax.axis_index("s")
    wid = cid * NS + sid
    base = wid * RW
    my_slice = sid * TG * D

    zeros16 = jnp.zeros((16,), jnp.float32)

    for k in range(16 * D // 16):
        zbuf[pl.ds(k * 16, 16)] = zeros16
    for r in range(16):
        for k in range(NV):
            zbuf2[r, pl.ds(k * 16, 16)] = zeros16
    for i in range(TG // 16):
        pltpu.sync_copy(zbuf, slices.at[pl.ds(my_slice + i * 16 * D, 16 * D)])
    pltpu.sync_copy(zbuf2, acc_sc.at[pl.ds(sid * 16, 16)])
    plsc.subcore_barrier()

    pltpu.sync_copy(z_hbm.at[pl.ds(base, RW)], z_v.at[pl.ds(0, RW)])
    pltpu.sync_copy(b_hbm.at[pl.ds(base, RW)], b_v.at[pl.ds(0, RW)])

    def start_row(j, b):
        pltpu.make_async_copy(
            x_hbm.at[pl.ds(base + j * C, C)], rows[b], sem_row[b]).start()

    def wait_row(b):
        pltpu.make_async_copy(
            x_hbm.at[pl.ds(0, C)], rows[b], sem_row[b]).wait()

    def flush(g_cur, tot, vrt):
        for k in range(NV):
            stage[pl.ds(k * 16, 16)] = tot[k] - vrt[k]
            stage[pl.ds(D + k * 16, 16)] = vrt[k]
        pltpu.sync_copy(stage, slices.at[pl.ds(my_slice + 2 * g_cur * D, 2 * D)])

    for i in range(PF):
        start_row(i, i)

    def chunk_body(j, b, carry):
        g_cur = carry[0]
        tot = list(carry[1:1 + NV])
        vrt = list(carry[1 + NV:])
        goff = j * C
        g_first = b_v[pl.ds(goff, 16)][0]
        g_last = b_v[pl.ds(goff + C - 16, 16)][15]
        uniform = g_first == g_last
        reset = jnp.logical_or(g_first != g_cur, jnp.logical_not(uniform))

        @pl.when(reset)
        def _(g_cur=g_cur, tot=tuple(tot), vrt=tuple(vrt)):
            flush(g_cur, tot, vrt)

        def grp(q, gc):
            ct = list(gc[:NV])
            cv = list(gc[NV:])
            zvq = z_v[pl.ds(goff + q * 16, 16)]
            mfv = jnp.where(zvq == VIRT, 1.0, 0.0)
            for r in range(16):
                lane = jnp.broadcast_to(jnp.int32(r), (16,))
                mfb = mfv.at[lane].get(mode="promise_in_bounds")
                for k in range(NV):
                    rk = rows[b][q * 16 + r, pl.ds(k * 16, 16)]
                    ct[k] = ct[k] + rk
                    cv[k] = cv[k] + rk * mfb
            return (*ct, *cv)

        csum = lax.fori_loop(0, NQ, grp, (zeros16,) * (2 * NV))
        ctot = csum[:NV]
        cvrt = csum[NV:]

        @pl.when(jnp.logical_not(uniform))
        def _():
            for kk in range(NQ):
                zk = z_v[pl.ds(goff + kk * 16, 16)]
                bk = b_v[pl.ds(goff + kk * 16, 16)]
                dk = bk * 2 + jnp.where(zk == VIRT, 1, 0).astype(jnp.int32)
                dsts[pl.ds(kk * 16, 16)] = dk
            pltpu.sync_copy(rows[b], acc_sc.at[dsts], add=True)

        keep = jnp.where(reset, 0.0, 1.0)
        inc = jnp.where(uniform, 1.0, 0.0)
        new_tot = [tot[k] * keep + ctot[k] * inc for k in range(NV)]
        new_vrt = [vrt[k] * keep + cvrt[k] * inc for k in range(NV)]
        g_new = jnp.where(uniform, g_first, g_last)
        return (g_new, *new_tot, *new_vrt)

    init = (b_v[pl.ds(0, 16)][0],) + (zeros16,) * (2 * NV)

    def body(t, carry):
        for b in range(NBUF):
            j = NBUF * t + b
            wait_row(b)
            nb = (b + PF) % NBUF

            @pl.when(j + PF < NCH)
            def _():
                start_row(j + PF, nb)

            carry = chunk_body(j, b, carry)
        return carry

    fin = lax.fori_loop(0, NCH // NBUF, body, init)
    flush(fin[0], fin[1:1 + NV], fin[1 + NV:])

    plsc.subcore_barrier()

    rbase = sid * 16 * D
    pltpu.sync_copy(slices.at[pl.ds(rbase, 16 * D)], red)

    def red_body(w, carry):
        pltpu.sync_copy(slices.at[pl.ds(w * TG * D + rbase, 16 * D)], tmp)
        for k in range(16 * D // 16):
            red[pl.ds(k * 16, 16)] = (
                red[pl.ds(k * 16, 16)] + tmp[pl.ds(k * 16, 16)])
        return carry

    lax.fori_loop(1, NS, red_body, 0)

    pltpu.sync_copy(acc_sc.at[pl.ds(sid * 16, 16)], tmp2)
    for r in range(16):
        for k in range(NV):
            red[pl.ds(r * D + k * 16, 16)] = (
                red[pl.ds(r * D + k * 16, 16)] + tmp2[r, pl.ds(k * 16, 16)])

    pltpu.sync_copy(red, out_hbm.at[pl.ds(cid * TG * D + rbase, 16 * D)])


def _tc_body(x_ref, z_ref, b_ref, out_ref):
    i = pl.program_id(0)
    bb = jnp.broadcast_to(b_ref[0], (G, RTC))
    vv = jnp.broadcast_to(z_ref[0] == VIRT, (G, RTC))
    gi = lax.broadcasted_iota(jnp.int32, (G, RTC), 0)
    eq = gi == bb
    sreal = (eq & jnp.logical_not(vv)).astype(jnp.bfloat16)
    svirt = (eq & vv).astype(jnp.bfloat16)
    rb = x_ref[...].astype(jnp.bfloat16)
    dn = (((1,), (0,)), ((), ()))
    pr = lax.dot_general(sreal, rb, dn, preferred_element_type=jnp.float32)
    pv = lax.dot_general(svirt, rb, dn, preferred_element_type=jnp.float32)

    @pl.when(i == 0)
    def _():
        out_ref[0] = pr
        out_ref[1] = pv

    @pl.when(i > 0)
    def _():
        out_ref[0] += pr
        out_ref[1] += pv


_tc_pool = pl.pallas_call(
    _tc_body,
    grid=(NBT,),
    in_specs=[
        pl.BlockSpec((RTC, D), lambda i: (TC0 + i, 0)),
        pl.BlockSpec((1, 1, RTC), lambda i: (TC0 + i, 0, 0)),
        pl.BlockSpec((1, 1, RTC), lambda i: (TC0 + i, 0, 0)),
    ],
    out_specs=pl.BlockSpec((2, G, D), lambda i: (0, 0, 0)),
    out_shape=jax.ShapeDtypeStruct((2, G, D), jnp.float32),
    compiler_params=pltpu.CompilerParams(
        dimension_semantics=("arbitrary",)),
)


_tc_full = pl.pallas_call(
    _tc_body,
    grid=(N // RTC,),
    in_specs=[
        pl.BlockSpec((RTC, D), lambda i: (i, 0)),
        pl.BlockSpec((1, 1, RTC), lambda i: (i, 0, 0)),
        pl.BlockSpec((1, 1, RTC), lambda i: (i, 0, 0)),
    ],
    out_specs=pl.BlockSpec((2, G, D), lambda i: (0, 0, 0)),
    out_shape=jax.ShapeDtypeStruct((2, G, D), jnp.float32),
    compiler_params=pltpu.CompilerParams(
        dimension_semantics=("arbitrary",)),
)


def kernel(out, z_rv, x_rv_batch):
    z32 = z_rv.astype(jnp.int32)
    b32 = x_rv_batch.astype(jnp.int32)
    tc = _tc_full(out, z32.reshape(N // RTC, 1, RTC),
                  b32.reshape(N // RTC, 1, RTC))
    return jnp.concatenate((tc[0], tc[1]), axis=1)


# speedup vs baseline: 1.3717x; 1.3717x over previous
import functools

import jax
import jax.numpy as jnp
from jax import lax
from jax.experimental import pallas as pl
from jax.experimental.pallas import tpu as pltpu
from jax.experimental.pallas import tpu_sc as plsc

N = 320000
D = 128
G = 128
VIRT = 100
NC = 2
NS = 16
NW = NC * NS
NV = D // 16
TG = 2 * G

NSC = 102400
RW = NSC // NW
C = 80
NCH = RW // C
NBUF = 5
PF = 4
NQ = C // 16

RTC = 1280
NBT = (N - NSC) // RTC
TC0 = NSC // RTC


@functools.partial(
    pl.kernel,
    mesh=plsc.VectorSubcoreMesh(core_axis_name="c", subcore_axis_name="s"),
    out_type=jax.ShapeDtypeStruct((NC * TG * D,), jnp.float32),
    scratch_types=(
        [pltpu.VMEM((RW + 16,), jnp.int32),
         pltpu.VMEM((RW + 16,), jnp.int32)]
        + [pltpu.VMEM((C, D), jnp.float32)] * NBUF
        + [pltpu.VMEM((2 * D,), jnp.float32),
           pltpu.VMEM((16 * D,), jnp.float32),
           pltpu.VMEM((16, D), jnp.float32),
           pltpu.VMEM((16 * D,), jnp.float32),
           pltpu.VMEM((16 * D,), jnp.float32),
           pltpu.VMEM((16, D), jnp.float32),
           pltpu.VMEM((C,), jnp.int32),
           pltpu.VMEM_SHARED((NS * TG * D,), jnp.float32),
           pltpu.VMEM_SHARED((TG, D), jnp.float32)]
        + [pltpu.SemaphoreType.DMA] * NBUF
    ),
)
def _pool_kernel(x_hbm, z_hbm, b_hbm, out_hbm, z_v, b_v, *refs):
    rows = refs[0:NBUF]
    stage = refs[NBUF]
    zbuf = refs[NBUF + 1]
    zbuf2 = refs[NBUF + 2]
    red = refs[NBUF + 3]
    tmp = refs[NBUF + 4]
    tmp2 = refs[NBUF + 5]
    dsts = refs[NBUF + 6]
    slices = refs[NBUF + 7]
    acc_sc = refs[NBUF + 8]
    sem_row = refs[NBUF + 9:NBUF + 9 + NBUF]

    cid = lax.axis_index("c")
    sid = lax.axis_index("s")
    wid = cid * NS + sid
    base = wid * RW
    my_slice = sid * TG * D

    zeros16 = jnp.zeros((16,), jnp.float32)

    for k in range(16 * D // 16):
        zbuf[pl.ds(k * 16, 16)] = zeros16
    for r in range(16):
        for k in range(NV):
            zbuf2[r, pl.ds(k * 16, 16)] = zeros16
    for i in range(TG // 16):
        pltpu.sync_copy(zbuf, slices.at[pl.ds(my_slice + i * 16 * D, 16 * D)])
    pltpu.sync_copy(zbuf2, acc_sc.at[pl.ds(sid * 16, 16)])
    plsc.subcore_barrier()

    pltpu.sync_copy(z_hbm.at[pl.ds(base, RW)], z_v.at[pl.ds(0, RW)])
    pltpu.sync_copy(b_hbm.at[pl.ds(base, RW)], b_v.at[pl.ds(0, RW)])

    def start_row(j, b):
        pltpu.make_async_copy(
            x_hbm.at[pl.ds(base + j * C, C)], rows[b], sem_row[b]).start()

    def wait_row(b):
        pltpu.make_async_copy(
            x_hbm.at[pl.ds(0, C)], rows[b], sem_row[b]).wait()

    def flush(g_cur, tot, vrt):
        for k in range(NV):
            stage[pl.ds(k * 16, 16)] = tot[k] - vrt[k]
            stage[pl.ds(D + k * 16, 16)] = vrt[k]
        pltpu.sync_copy(stage, slices.at[pl.ds(my_slice + 2 * g_cur * D, 2 * D)])

    for i in range(PF):
        start_row(i, i)

    def chunk_body(j, b, carry):
        g_cur = carry[0]
        tot = list(carry[1:1 + NV])
        vrt = list(carry[1 + NV:])
        goff = j * C
        g_first = b_v[pl.ds(goff, 16)][0]
        g_last = b_v[pl.ds(goff + C - 16, 16)][15]
        uniform = g_first == g_last
        reset = jnp.logical_or(g_first != g_cur, jnp.logical_not(uniform))

        @pl.when(reset)
        def _(g_cur=g_cur, tot=tuple(tot), vrt=tuple(vrt)):
            flush(g_cur, tot, vrt)

        def grp(q, gc):
            ct = list(gc[:NV])
            cv = list(gc[NV:])
            zvq = z_v[pl.ds(goff + q * 16, 16)]
            mfv = jnp.where(zvq == VIRT, 1.0, 0.0)
            for r in range(16):
                lane = jnp.broadcast_to(jnp.int32(r), (16,))
                mfb = mfv.at[lane].get(mode="promise_in_bounds")
                for k in range(NV):
                    rk = rows[b][q * 16 + r, pl.ds(k * 16, 16)]
                    ct[k] = ct[k] + rk
                    cv[k] = cv[k] + rk * mfb
            return (*ct, *cv)

        csum = lax.fori_loop(0, NQ, grp, (zeros16,) * (2 * NV))
        ctot = csum[:NV]
        cvrt = csum[NV:]

        @pl.when(jnp.logical_not(uniform))
        def _():
            for kk in range(NQ):
                zk = z_v[pl.ds(goff + kk * 16, 16)]
                bk = b_v[pl.ds(goff + kk * 16, 16)]
                dk = bk * 2 + jnp.where(zk == VIRT, 1, 0).astype(jnp.int32)
                dsts[pl.ds(kk * 16, 16)] = dk
            pltpu.sync_copy(rows[b], acc_sc.at[dsts], add=True)

        keep = jnp.where(reset, 0.0, 1.0)
        inc = jnp.where(uniform, 1.0, 0.0)
        new_tot = [tot[k] * keep + ctot[k] * inc for k in range(NV)]
        new_vrt = [vrt[k] * keep + cvrt[k] * inc for k in range(NV)]
        g_new = jnp.where(uniform, g_first, g_last)
        return (g_new, *new_tot, *new_vrt)

    init = (b_v[pl.ds(0, 16)][0],) + (zeros16,) * (2 * NV)

    def body(t, carry):
        for b in range(NBUF):
            j = NBUF * t + b
            wait_row(b)
            nb = (b + PF) % NBUF

            @pl.when(j + PF < NCH)
            def _():
                start_row(j + PF, nb)

            carry = chunk_body(j, b, carry)
        return carry

    fin = lax.fori_loop(0, NCH // NBUF, body, init)
    flush(fin[0], fin[1:1 + NV], fin[1 + NV:])

    plsc.subcore_barrier()

    rbase = sid * 16 * D
    pltpu.sync_copy(slices.at[pl.ds(rbase, 16 * D)], red)

    def red_body(w, carry):
        pltpu.sync_copy(slices.at[pl.ds(w * TG * D + rbase, 16 * D)], tmp)
        for k in range(16 * D // 16):
            red[pl.ds(k * 16, 16)] = (
                red[pl.ds(k * 16, 16)] + tmp[pl.ds(k * 16, 16)])
        return carry

    lax.fori_loop(1, NS, red_body, 0)

    pltpu.sync_copy(acc_sc.at[pl.ds(sid * 16, 16)], tmp2)
    for r in range(16):
        for k in range(NV):
            red[pl.ds(r * D + k * 16, 16)] = (
                red[pl.ds(r * D + k * 16, 16)] + tmp2[r, pl.ds(k * 16, 16)])

    pltpu.sync_copy(red, out_hbm.at[pl.ds(cid * TG * D + rbase, 16 * D)])


def _tc_body(x_ref, z_ref, b_ref, out_ref):
    i = pl.program_id(0)

    @pl.when(i == 0)
    def _():
        out_ref[...] = jnp.zeros((2, G, D), jnp.float32)

    g_min = b_ref[0, 0, 0]
    g_max = b_ref[0, 0, RTC - 1]
    gb = jnp.minimum((g_min // 8) * 8, G - 16)
    fast = g_max < gb + 16

    zrow = z_ref[0]
    brow = b_ref[0]
    vmask = zrow == VIRT
    x = x_ref[...]
    dn = (((1,), (0,)), ((), ()))

    @pl.when(fast)
    def _():
        bb = jnp.broadcast_to(brow, (16, RTC))
        vv = jnp.broadcast_to(vmask, (16, RTC))
        gi = lax.broadcasted_iota(jnp.int32, (16, RTC), 0) + gb
        eq = gi == bb
        sreal = (eq & jnp.logical_not(vv)).astype(jnp.float32)
        svirt = (eq & vv).astype(jnp.float32)
        pr = lax.dot_general(sreal, x, dn, preferred_element_type=jnp.float32)
        pv = lax.dot_general(svirt, x, dn, preferred_element_type=jnp.float32)
        out_ref[0, pl.ds(gb, 16), :] += pr
        out_ref[1, pl.ds(gb, 16), :] += pv

    @pl.when(jnp.logical_not(fast))
    def _():
        bb = jnp.broadcast_to(brow, (G, RTC))
        vv = jnp.broadcast_to(vmask, (G, RTC))
        gi = lax.broadcasted_iota(jnp.int32, (G, RTC), 0)
        eq = gi == bb
        sreal = (eq & jnp.logical_not(vv)).astype(jnp.float32)
        svirt = (eq & vv).astype(jnp.float32)
        pr = lax.dot_general(sreal, x, dn, preferred_element_type=jnp.float32)
        pv = lax.dot_general(svirt, x, dn, preferred_element_type=jnp.float32)
        out_ref[0] += pr
        out_ref[1] += pv


_tc_pool = pl.pallas_call(
    _tc_body,
    grid=(NBT,),
    in_specs=[
        pl.BlockSpec((RTC, D), lambda i: (TC0 + i, 0)),
        pl.BlockSpec((1, 1, RTC), lambda i: (TC0 + i, 0, 0)),
        pl.BlockSpec((1, 1, RTC), lambda i: (TC0 + i, 0, 0)),
    ],
    out_specs=pl.BlockSpec((2, G, D), lambda i: (0, 0, 0)),
    out_shape=jax.ShapeDtypeStruct((2, G, D), jnp.float32),
    compiler_params=pltpu.CompilerParams(
        dimension_semantics=("arbitrary",)),
)


def kernel(out, z_rv, x_rv_batch):
    z32 = z_rv.astype(jnp.int32)
    b32 = x_rv_batch.astype(jnp.int32)
    sc = _pool_kernel(out, z32, b32)
    tc = _tc_pool(out, z32.reshape(N // RTC, 1, RTC),
                  b32.reshape(N // RTC, 1, RTC))
    scp = (sc.reshape(NC, TG * D).sum(0)).reshape(G, 2, D)
    real = scp[:, 0, :] + tc[0]
    virt = scp[:, 1, :] + tc[1]
    return jnp.concatenate((real, virt), axis=1)


# speedup vs baseline: 1.9641x; 1.4319x over previous
import functools

import jax
import jax.numpy as jnp
from jax import lax
from jax.experimental import pallas as pl
from jax.experimental.pallas import tpu as pltpu
from jax.experimental.pallas import tpu_sc as plsc

N = 320000
D = 128
G = 128
VIRT = 100
NC = 2
NS = 16
NW = NC * NS
NV = D // 16
TG = 2 * G

NSC = 102400
RW = NSC // NW
C = 80
NCH = RW // C
NBUF = 5
PF = 4
NQ = C // 16

RTC = 2560
NBT = (N - NSC) // RTC
TC0 = NSC // RTC


@functools.partial(
    pl.kernel,
    mesh=plsc.VectorSubcoreMesh(core_axis_name="c", subcore_axis_name="s"),
    out_type=jax.ShapeDtypeStruct((NC * TG * D,), jnp.float32),
    scratch_types=(
        [pltpu.VMEM((RW + 16,), jnp.int32),
         pltpu.VMEM((RW + 16,), jnp.int32)]
        + [pltpu.VMEM((C, D), jnp.float32)] * NBUF
        + [pltpu.VMEM((2 * D,), jnp.float32),
           pltpu.VMEM((16 * D,), jnp.float32),
           pltpu.VMEM((16, D), jnp.float32),
           pltpu.VMEM((16 * D,), jnp.float32),
           pltpu.VMEM((16 * D,), jnp.float32),
           pltpu.VMEM((16, D), jnp.float32),
           pltpu.VMEM((C,), jnp.int32),
           pltpu.VMEM_SHARED((NS * TG * D,), jnp.float32),
           pltpu.VMEM_SHARED((TG, D), jnp.float32)]
        + [pltpu.SemaphoreType.DMA] * NBUF
    ),
)
def _pool_kernel(x_hbm, z_hbm, b_hbm, out_hbm, z_v, b_v, *refs):
    rows = refs[0:NBUF]
    stage = refs[NBUF]
    zbuf = refs[NBUF + 1]
    zbuf2 = refs[NBUF + 2]
    red = refs[NBUF + 3]
    tmp = refs[NBUF + 4]
    tmp2 = refs[NBUF + 5]
    dsts = refs[NBUF + 6]
    slices = refs[NBUF + 7]
    acc_sc = refs[NBUF + 8]
    sem_row = refs[NBUF + 9:NBUF + 9 + NBUF]

    cid = lax.axis_index("c")
    sid = lax.axis_index("s")
    wid = cid * NS + sid
    base = wid * RW
    my_slice = sid * TG * D

    zeros16 = jnp.zeros((16,), jnp.float32)

    for k in range(16 * D // 16):
        zbuf[pl.ds(k * 16, 16)] = zeros16
    for r in range(16):
        for k in range(NV):
            zbuf2[r, pl.ds(k * 16, 16)] = zeros16
    for i in range(TG // 16):
        pltpu.sync_copy(zbuf, slices.at[pl.ds(my_slice + i * 16 * D, 16 * D)])
    pltpu.sync_copy(zbuf2, acc_sc.at[pl.ds(sid * 16, 16)])
    plsc.subcore_barrier()

    pltpu.sync_copy(z_hbm.at[pl.ds(base, RW)], z_v.at[pl.ds(0, RW)])
    pltpu.sync_copy(b_hbm.at[pl.ds(base, RW)], b_v.at[pl.ds(0, RW)])

    def start_row(j, b):
        pltpu.make_async_copy(
            x_hbm.at[pl.ds(base + j * C, C)], rows[b], sem_row[b]).start()

    def wait_row(b):
        pltpu.make_async_copy(
            x_hbm.at[pl.ds(0, C)], rows[b], sem_row[b]).wait()

    def flush(g_cur, tot, vrt):
        for k in range(NV):
            stage[pl.ds(k * 16, 16)] = tot[k] - vrt[k]
            stage[pl.ds(D + k * 16, 16)] = vrt[k]
        pltpu.sync_copy(stage, slices.at[pl.ds(my_slice + 2 * g_cur * D, 2 * D)])

    for i in range(PF):
        start_row(i, i)

    def chunk_body(j, b, carry):
        g_cur = carry[0]
        tot = list(carry[1:1 + NV])
        vrt = list(carry[1 + NV:])
        goff = j * C
        g_first = b_v[pl.ds(goff, 16)][0]
        g_last = b_v[pl.ds(goff + C - 16, 16)][15]
        uniform = g_first == g_last
        reset = jnp.logical_or(g_first != g_cur, jnp.logical_not(uniform))

        @pl.when(reset)
        def _(g_cur=g_cur, tot=tuple(tot), vrt=tuple(vrt)):
            flush(g_cur, tot, vrt)

        def grp(q, gc):
            ct = list(gc[:NV])
            cv = list(gc[NV:])
            zvq = z_v[pl.ds(goff + q * 16, 16)]
            mfv = jnp.where(zvq == VIRT, 1.0, 0.0)
            for r in range(16):
                lane = jnp.broadcast_to(jnp.int32(r), (16,))
                mfb = mfv.at[lane].get(mode="promise_in_bounds")
                for k in range(NV):
                    rk = rows[b][q * 16 + r, pl.ds(k * 16, 16)]
                    ct[k] = ct[k] + rk
                    cv[k] = cv[k] + rk * mfb
            return (*ct, *cv)

        csum = lax.fori_loop(0, NQ, grp, (zeros16,) * (2 * NV))
        ctot = csum[:NV]
        cvrt = csum[NV:]

        @pl.when(jnp.logical_not(uniform))
        def _():
            for kk in range(NQ):
                zk = z_v[pl.ds(goff + kk * 16, 16)]
                bk = b_v[pl.ds(goff + kk * 16, 16)]
                dk = bk * 2 + jnp.where(zk == VIRT, 1, 0).astype(jnp.int32)
                dsts[pl.ds(kk * 16, 16)] = dk
            pltpu.sync_copy(rows[b], acc_sc.at[dsts], add=True)

        keep = jnp.where(reset, 0.0, 1.0)
        inc = jnp.where(uniform, 1.0, 0.0)
        new_tot = [tot[k] * keep + ctot[k] * inc for k in range(NV)]
        new_vrt = [vrt[k] * keep + cvrt[k] * inc for k in range(NV)]
        g_new = jnp.where(uniform, g_first, g_last)
        return (g_new, *new_tot, *new_vrt)

    init = (b_v[pl.ds(0, 16)][0],) + (zeros16,) * (2 * NV)

    def body(t, carry):
        for b in range(NBUF):
            j = NBUF * t + b
            wait_row(b)
            nb = (b + PF) % NBUF

            @pl.when(j + PF < NCH)
            def _():
                start_row(j + PF, nb)

            carry = chunk_body(j, b, carry)
        return carry

    fin = lax.fori_loop(0, NCH // NBUF, body, init)
    flush(fin[0], fin[1:1 + NV], fin[1 + NV:])

    plsc.subcore_barrier()

    rbase = sid * 16 * D
    pltpu.sync_copy(slices.at[pl.ds(rbase, 16 * D)], red)

    def red_body(w, carry):
        pltpu.sync_copy(slices.at[pl.ds(w * TG * D + rbase, 16 * D)], tmp)
        for k in range(16 * D // 16):
            red[pl.ds(k * 16, 16)] = (
                red[pl.ds(k * 16, 16)] + tmp[pl.ds(k * 16, 16)])
        return carry

    lax.fori_loop(1, NS, red_body, 0)

    pltpu.sync_copy(acc_sc.at[pl.ds(sid * 16, 16)], tmp2)
    for r in range(16):
        for k in range(NV):
            red[pl.ds(r * D + k * 16, 16)] = (
                red[pl.ds(r * D + k * 16, 16)] + tmp2[r, pl.ds(k * 16, 16)])

    pltpu.sync_copy(red, out_hbm.at[pl.ds(cid * TG * D + rbase, 16 * D)])


def _tc_body(x_ref, z_ref, b_ref, out_ref):
    i = pl.program_id(0)

    @pl.when(i == 0)
    def _():
        out_ref[...] = jnp.zeros((2, G, D), jnp.float32)

    g_min = b_ref[0, 0, 0]
    g_max = b_ref[0, 0, RTC - 1]
    gb = jnp.minimum((g_min // 8) * 8, G - 16)
    fast = g_max < gb + 16

    zrow = z_ref[0]
    brow = b_ref[0]
    vmask = zrow == VIRT
    x = x_ref[...]
    dn = (((1,), (0,)), ((), ()))

    @pl.when(fast)
    def _():
        bb = jnp.broadcast_to(brow, (16, RTC))
        vv = jnp.broadcast_to(vmask, (16, RTC))
        gi = lax.broadcasted_iota(jnp.int32, (16, RTC), 0) + gb
        eq = gi == bb
        sreal = (eq & jnp.logical_not(vv)).astype(jnp.float32)
        svirt = (eq & vv).astype(jnp.float32)
        pr = lax.dot_general(sreal, x, dn, preferred_element_type=jnp.float32)
        pv = lax.dot_general(svirt, x, dn, preferred_element_type=jnp.float32)
        out_ref[0, pl.ds(gb, 16), :] += pr
        out_ref[1, pl.ds(gb, 16), :] += pv

    @pl.when(jnp.logical_not(fast))
    def _():
        bb = jnp.broadcast_to(brow, (G, RTC))
        vv = jnp.broadcast_to(vmask, (G, RTC))
        gi = lax.broadcasted_iota(jnp.int32, (G, RTC), 0)
        eq = gi == bb
        sreal = (eq & jnp.logical_not(vv)).astype(jnp.float32)
        svirt = (eq & vv).astype(jnp.float32)
        pr = lax.dot_general(sreal, x, dn, preferred_element_type=jnp.float32)
        pv = lax.dot_general(svirt, x, dn, preferred_element_type=jnp.float32)
        out_ref[0] += pr
        out_ref[1] += pv


_tc_pool = pl.pallas_call(
    _tc_body,
    grid=(NBT,),
    in_specs=[
        pl.BlockSpec((RTC, D), lambda i: (TC0 + i, 0)),
        pl.BlockSpec((1, 1, RTC), lambda i: (TC0 + i, 0, 0)),
        pl.BlockSpec((1, 1, RTC), lambda i: (TC0 + i, 0, 0)),
    ],
    out_specs=pl.BlockSpec((2, G, D), lambda i: (0, 0, 0)),
    out_shape=jax.ShapeDtypeStruct((2, G, D), jnp.float32),
    compiler_params=pltpu.CompilerParams(
        dimension_semantics=("arbitrary",)),
)


def kernel(out, z_rv, x_rv_batch):
    z32 = z_rv.astype(jnp.int32)
    b32 = x_rv_batch.astype(jnp.int32)
    sc = _pool_kernel(out, z32, b32)
    tc = _tc_pool(out, z32.reshape(N // RTC, 1, RTC),
                  b32.reshape(N // RTC, 1, RTC))
    scp = (sc.reshape(NC, TG * D).sum(0)).reshape(G, 2, D)
    real = scp[:, 0, :] + tc[0]
    virt = scp[:, 1, :] + tc[1]
    return jnp.concatenate((real, virt), axis=1)


# speedup vs baseline: 2.5289x; 1.2875x over previous
import functools

import jax
import jax.numpy as jnp
from jax import lax
from jax.experimental import pallas as pl
from jax.experimental.pallas import tpu as pltpu
from jax.experimental.pallas import tpu_sc as plsc

N = 320000
D = 128
G = 128
VIRT = 100
NC = 2
NS = 16
NW = NC * NS
NV = D // 16
TG = 2 * G

NSC = 102400
RW = NSC // NW
C = 80
NCH = RW // C
NBUF = 5
PF = 4
NQ = C // 16

RTC = 6400
NBT = (N - NSC) // RTC
TC0 = NSC // RTC


@functools.partial(
    pl.kernel,
    mesh=plsc.VectorSubcoreMesh(core_axis_name="c", subcore_axis_name="s"),
    out_type=jax.ShapeDtypeStruct((NC * TG * D,), jnp.float32),
    scratch_types=(
        [pltpu.VMEM((RW + 16,), jnp.int32),
         pltpu.VMEM((RW + 16,), jnp.int32)]
        + [pltpu.VMEM((C, D), jnp.float32)] * NBUF
        + [pltpu.VMEM((2 * D,), jnp.float32),
           pltpu.VMEM((16 * D,), jnp.float32),
           pltpu.VMEM((16, D), jnp.float32),
           pltpu.VMEM((16 * D,), jnp.float32),
           pltpu.VMEM((16 * D,), jnp.float32),
           pltpu.VMEM((16, D), jnp.float32),
           pltpu.VMEM((C,), jnp.int32),
           pltpu.VMEM_SHARED((NS * TG * D,), jnp.float32),
           pltpu.VMEM_SHARED((TG, D), jnp.float32)]
        + [pltpu.SemaphoreType.DMA] * NBUF
    ),
)
def _pool_kernel(x_hbm, z_hbm, b_hbm, out_hbm, z_v, b_v, *refs):
    rows = refs[0:NBUF]
    stage = refs[NBUF]
    zbuf = refs[NBUF + 1]
    zbuf2 = refs[NBUF + 2]
    red = refs[NBUF + 3]
    tmp = refs[NBUF + 4]
    tmp2 = refs[NBUF + 5]
    dsts = refs[NBUF + 6]
    slices = refs[NBUF + 7]
    acc_sc = refs[NBUF + 8]
    sem_row = refs[NBUF + 9:NBUF + 9 + NBUF]

    cid = lax.axis_index("c")
    sid = lax.axis_index("s")
    wid = cid * NS + sid
    base = wid * RW
    my_slice = sid * TG * D

    zeros16 = jnp.zeros((16,), jnp.float32)

    for k in range(16 * D // 16):
        zbuf[pl.ds(k * 16, 16)] = zeros16
    for r in range(16):
        for k in range(NV):
            zbuf2[r, pl.ds(k * 16, 16)] = zeros16
    for i in range(TG // 16):
        pltpu.sync_copy(zbuf, slices.at[pl.ds(my_slice + i * 16 * D, 16 * D)])
    pltpu.sync_copy(zbuf2, acc_sc.at[pl.ds(sid * 16, 16)])
    plsc.subcore_barrier()

    pltpu.sync_copy(z_hbm.at[pl.ds(base, RW)], z_v.at[pl.ds(0, RW)])
    pltpu.sync_copy(b_hbm.at[pl.ds(base, RW)], b_v.at[pl.ds(0, RW)])

    def start_row(j, b):
        pltpu.make_async_copy(
            x_hbm.at[pl.ds(base + j * C, C)], rows[b], sem_row[b]).start()

    def wait_row(b):
        pltpu.make_async_copy(
            x_hbm.at[pl.ds(0, C)], rows[b], sem_row[b]).wait()

    def flush(g_cur, tot, vrt):
        for k in range(NV):
            stage[pl.ds(k * 16, 16)] = tot[k] - vrt[k]
            stage[pl.ds(D + k * 16, 16)] = vrt[k]
        pltpu.sync_copy(stage, slices.at[pl.ds(my_slice + 2 * g_cur * D, 2 * D)])

    for i in range(PF):
        start_row(i, i)

    def chunk_body(j, b, carry):
        g_cur = carry[0]
        tot = list(carry[1:1 + NV])
        vrt = list(carry[1 + NV:])
        goff = j * C
        g_first = b_v[pl.ds(goff, 16)][0]
        g_last = b_v[pl.ds(goff + C - 16, 16)][15]
        uniform = g_first == g_last
        reset = jnp.logical_or(g_first != g_cur, jnp.logical_not(uniform))

        @pl.when(reset)
        def _(g_cur=g_cur, tot=tuple(tot), vrt=tuple(vrt)):
            flush(g_cur, tot, vrt)

        def grp(q, gc):
            ct = list(gc[:NV])
            cv = list(gc[NV:])
            zvq = z_v[pl.ds(goff + q * 16, 16)]
            mfv = jnp.where(zvq == VIRT, 1.0, 0.0)
            for r in range(16):
                lane = jnp.broadcast_to(jnp.int32(r), (16,))
                mfb = mfv.at[lane].get(mode="promise_in_bounds")
                for k in range(NV):
                    rk = rows[b][q * 16 + r, pl.ds(k * 16, 16)]
                    ct[k] = ct[k] + rk
                    cv[k] = cv[k] + rk * mfb
            return (*ct, *cv)

        csum = lax.fori_loop(0, NQ, grp, (zeros16,) * (2 * NV))
        ctot = csum[:NV]
        cvrt = csum[NV:]

        @pl.when(jnp.logical_not(uniform))
        def _():
            for kk in range(NQ):
                zk = z_v[pl.ds(goff + kk * 16, 16)]
                bk = b_v[pl.ds(goff + kk * 16, 16)]
                dk = bk * 2 + jnp.where(zk == VIRT, 1, 0).astype(jnp.int32)
                dsts[pl.ds(kk * 16, 16)] = dk
            pltpu.sync_copy(rows[b], acc_sc.at[dsts], add=True)

        keep = jnp.where(reset, 0.0, 1.0)
        inc = jnp.where(uniform, 1.0, 0.0)
        new_tot = [tot[k] * keep + ctot[k] * inc for k in range(NV)]
        new_vrt = [vrt[k] * keep + cvrt[k] * inc for k in range(NV)]
        g_new = jnp.where(uniform, g_first, g_last)
        return (g_new, *new_tot, *new_vrt)

    init = (b_v[pl.ds(0, 16)][0],) + (zeros16,) * (2 * NV)

    def body(t, carry):
        for b in range(NBUF):
            j = NBUF * t + b
            wait_row(b)
            nb = (b + PF) % NBUF

            @pl.when(j + PF < NCH)
            def _():
                start_row(j + PF, nb)

            carry = chunk_body(j, b, carry)
        return carry

    fin = lax.fori_loop(0, NCH // NBUF, body, init)
    flush(fin[0], fin[1:1 + NV], fin[1 + NV:])

    plsc.subcore_barrier()

    rbase = sid * 16 * D
    pltpu.sync_copy(slices.at[pl.ds(rbase, 16 * D)], red)

    def red_body(w, carry):
        pltpu.sync_copy(slices.at[pl.ds(w * TG * D + rbase, 16 * D)], tmp)
        for k in range(16 * D // 16):
            red[pl.ds(k * 16, 16)] = (
                red[pl.ds(k * 16, 16)] + tmp[pl.ds(k * 16, 16)])
        return carry

    lax.fori_loop(1, NS, red_body, 0)

    pltpu.sync_copy(acc_sc.at[pl.ds(sid * 16, 16)], tmp2)
    for r in range(16):
        for k in range(NV):
            red[pl.ds(r * D + k * 16, 16)] = (
                red[pl.ds(r * D + k * 16, 16)] + tmp2[r, pl.ds(k * 16, 16)])

    pltpu.sync_copy(red, out_hbm.at[pl.ds(cid * TG * D + rbase, 16 * D)])


def _tc_body(x_ref, z_ref, b_ref, out_ref):
    i = pl.program_id(0)

    @pl.when(i == 0)
    def _():
        out_ref[...] = jnp.zeros((2, G, D), jnp.float32)

    g_min = b_ref[0, 0, 0]
    g_max = b_ref[0, 0, RTC - 1]
    gb = jnp.minimum((g_min // 8) * 8, G - 16)
    fast = g_max < gb + 16

    zrow = z_ref[0]
    brow = b_ref[0]
    vmask = zrow == VIRT
    x = x_ref[...]
    dn = (((1,), (0,)), ((), ()))

    @pl.when(fast)
    def _():
        bb = jnp.broadcast_to(brow, (16, RTC))
        vv = jnp.broadcast_to(vmask, (16, RTC))
        gi = lax.broadcasted_iota(jnp.int32, (16, RTC), 0) + gb
        eq = gi == bb
        sreal = (eq & jnp.logical_not(vv)).astype(jnp.float32)
        svirt = (eq & vv).astype(jnp.float32)
        pr = lax.dot_general(sreal, x, dn, preferred_element_type=jnp.float32)
        pv = lax.dot_general(svirt, x, dn, preferred_element_type=jnp.float32)
        out_ref[0, pl.ds(gb, 16), :] += pr
        out_ref[1, pl.ds(gb, 16), :] += pv

    @pl.when(jnp.logical_not(fast))
    def _():
        bb = jnp.broadcast_to(brow, (G, RTC))
        vv = jnp.broadcast_to(vmask, (G, RTC))
        gi = lax.broadcasted_iota(jnp.int32, (G, RTC), 0)
        eq = gi == bb
        sreal = (eq & jnp.logical_not(vv)).astype(jnp.float32)
        svirt = (eq & vv).astype(jnp.float32)
        pr = lax.dot_general(sreal, x, dn, preferred_element_type=jnp.float32)
        pv = lax.dot_general(svirt, x, dn, preferred_element_type=jnp.float32)
        out_ref[0] += pr
        out_ref[1] += pv


_tc_pool = pl.pallas_call(
    _tc_body,
    grid=(NBT,),
    in_specs=[
        pl.BlockSpec((RTC, D), lambda i: (TC0 + i, 0)),
        pl.BlockSpec((1, 1, RTC), lambda i: (TC0 + i, 0, 0)),
        pl.BlockSpec((1, 1, RTC), lambda i: (TC0 + i, 0, 0)),
    ],
    out_specs=pl.BlockSpec((2, G, D), lambda i: (0, 0, 0)),
    out_shape=jax.ShapeDtypeStruct((2, G, D), jnp.float32),
    compiler_params=pltpu.CompilerParams(
        dimension_semantics=("arbitrary",)),
)


def kernel(out, z_rv, x_rv_batch):
    z32 = z_rv.astype(jnp.int32)
    b32 = x_rv_batch.astype(jnp.int32)
    sc = _pool_kernel(out, z32, b32)
    tc = _tc_pool(out, z32.reshape(N // RTC, 1, RTC),
                  b32.reshape(N // RTC, 1, RTC))
    scp = (sc.reshape(NC, TG * D).sum(0)).reshape(G, 2, D)
    real = scp[:, 0, :] + tc[0]
    virt = scp[:, 1, :] + tc[1]
    return jnp.concatenate((real, virt), axis=1)


# speedup vs baseline: 2.6658x; 1.0541x over previous
import functools

import jax
import jax.numpy as jnp
from jax import lax
from jax.experimental import pallas as pl
from jax.experimental.pallas import tpu as pltpu
from jax.experimental.pallas import tpu_sc as plsc

N = 320000
D = 128
G = 128
VIRT = 100
NC = 2
NS = 16
NW = NC * NS
NV = D // 16
TG = 2 * G

NSC = 102400
RW = NSC // NW
C = 80
NCH = RW // C
NBUF = 5
PF = 4
NQ = C // 16

RTC = 12800
NBT = (N - NSC) // RTC
TC0 = NSC // RTC


@functools.partial(
    pl.kernel,
    mesh=plsc.VectorSubcoreMesh(core_axis_name="c", subcore_axis_name="s"),
    out_type=jax.ShapeDtypeStruct((NC * TG * D,), jnp.float32),
    scratch_types=(
        [pltpu.VMEM((RW + 16,), jnp.int32),
         pltpu.VMEM((RW + 16,), jnp.int32)]
        + [pltpu.VMEM((C, D), jnp.float32)] * NBUF
        + [pltpu.VMEM((2 * D,), jnp.float32),
           pltpu.VMEM((16 * D,), jnp.float32),
           pltpu.VMEM((16, D), jnp.float32),
           pltpu.VMEM((16 * D,), jnp.float32),
           pltpu.VMEM((16 * D,), jnp.float32),
           pltpu.VMEM((16, D), jnp.float32),
           pltpu.VMEM((C,), jnp.int32),
           pltpu.VMEM_SHARED((NS * TG * D,), jnp.float32),
           pltpu.VMEM_SHARED((TG, D), jnp.float32)]
        + [pltpu.SemaphoreType.DMA] * NBUF
    ),
)
def _pool_kernel(x_hbm, z_hbm, b_hbm, out_hbm, z_v, b_v, *refs):
    rows = refs[0:NBUF]
    stage = refs[NBUF]
    zbuf = refs[NBUF + 1]
    zbuf2 = refs[NBUF + 2]
    red = refs[NBUF + 3]
    tmp = refs[NBUF + 4]
    tmp2 = refs[NBUF + 5]
    dsts = refs[NBUF + 6]
    slices = refs[NBUF + 7]
    acc_sc = refs[NBUF + 8]
    sem_row = refs[NBUF + 9:NBUF + 9 + NBUF]

    cid = lax.axis_index("c")
    sid = lax.axis_index("s")
    wid = cid * NS + sid
    base = wid * RW
    my_slice = sid * TG * D

    zeros16 = jnp.zeros((16,), jnp.float32)

    for k in range(16 * D // 16):
        zbuf[pl.ds(k * 16, 16)] = zeros16
    for r in range(16):
        for k in range(NV):
            zbuf2[r, pl.ds(k * 16, 16)] = zeros16
    for i in range(TG // 16):
        pltpu.sync_copy(zbuf, slices.at[pl.ds(my_slice + i * 16 * D, 16 * D)])
    pltpu.sync_copy(zbuf2, acc_sc.at[pl.ds(sid * 16, 16)])
    plsc.subcore_barrier()

    pltpu.sync_copy(z_hbm.at[pl.ds(base, RW)], z_v.at[pl.ds(0, RW)])
    pltpu.sync_copy(b_hbm.at[pl.ds(base, RW)], b_v.at[pl.ds(0, RW)])

    def start_row(j, b):
        pltpu.make_async_copy(
            x_hbm.at[pl.ds(base + j * C, C)], rows[b], sem_row[b]).start()

    def wait_row(b):
        pltpu.make_async_copy(
            x_hbm.at[pl.ds(0, C)], rows[b], sem_row[b]).wait()

    def flush(g_cur, tot, vrt):
        for k in range(NV):
            stage[pl.ds(k * 16, 16)] = tot[k] - vrt[k]
            stage[pl.ds(D + k * 16, 16)] = vrt[k]
        pltpu.sync_copy(stage, slices.at[pl.ds(my_slice + 2 * g_cur * D, 2 * D)])

    for i in range(PF):
        start_row(i, i)

    def chunk_body(j, b, carry):
        g_cur = carry[0]
        tot = list(carry[1:1 + NV])
        vrt = list(carry[1 + NV:])
        goff = j * C
        g_first = b_v[pl.ds(goff, 16)][0]
        g_last = b_v[pl.ds(goff + C - 16, 16)][15]
        uniform = g_first == g_last
        reset = jnp.logical_or(g_first != g_cur, jnp.logical_not(uniform))

        @pl.when(reset)
        def _(g_cur=g_cur, tot=tuple(tot), vrt=tuple(vrt)):
            flush(g_cur, tot, vrt)

        def grp(q, gc):
            ct = list(gc[:NV])
            cv = list(gc[NV:])
            zvq = z_v[pl.ds(goff + q * 16, 16)]
            mfv = jnp.where(zvq == VIRT, 1.0, 0.0)
            for r in range(16):
                lane = jnp.broadcast_to(jnp.int32(r), (16,))
                mfb = mfv.at[lane].get(mode="promise_in_bounds")
                for k in range(NV):
                    rk = rows[b][q * 16 + r, pl.ds(k * 16, 16)]
                    ct[k] = ct[k] + rk
                    cv[k] = cv[k] + rk * mfb
            return (*ct, *cv)

        csum = lax.fori_loop(0, NQ, grp, (zeros16,) * (2 * NV))
        ctot = csum[:NV]
        cvrt = csum[NV:]

        @pl.when(jnp.logical_not(uniform))
        def _():
            for kk in range(NQ):
                zk = z_v[pl.ds(goff + kk * 16, 16)]
                bk = b_v[pl.ds(goff + kk * 16, 16)]
                dk = bk * 2 + jnp.where(zk == VIRT, 1, 0).astype(jnp.int32)
                dsts[pl.ds(kk * 16, 16)] = dk
            pltpu.sync_copy(rows[b], acc_sc.at[dsts], add=True)

        keep = jnp.where(reset, 0.0, 1.0)
        inc = jnp.where(uniform, 1.0, 0.0)
        new_tot = [tot[k] * keep + ctot[k] * inc for k in range(NV)]
        new_vrt = [vrt[k] * keep + cvrt[k] * inc for k in range(NV)]
        g_new = jnp.where(uniform, g_first, g_last)
        return (g_new, *new_tot, *new_vrt)

    init = (b_v[pl.ds(0, 16)][0],) + (zeros16,) * (2 * NV)

    def body(t, carry):
        for b in range(NBUF):
            j = NBUF * t + b
            wait_row(b)
            nb = (b + PF) % NBUF

            @pl.when(j + PF < NCH)
            def _():
                start_row(j + PF, nb)

            carry = chunk_body(j, b, carry)
        return carry

    fin = lax.fori_loop(0, NCH // NBUF, body, init)
    flush(fin[0], fin[1:1 + NV], fin[1 + NV:])

    plsc.subcore_barrier()

    rbase = sid * 16 * D
    pltpu.sync_copy(slices.at[pl.ds(rbase, 16 * D)], red)

    def red_body(w, carry):
        pltpu.sync_copy(slices.at[pl.ds(w * TG * D + rbase, 16 * D)], tmp)
        for k in range(16 * D // 16):
            red[pl.ds(k * 16, 16)] = (
                red[pl.ds(k * 16, 16)] + tmp[pl.ds(k * 16, 16)])
        return carry

    lax.fori_loop(1, NS, red_body, 0)

    pltpu.sync_copy(acc_sc.at[pl.ds(sid * 16, 16)], tmp2)
    for r in range(16):
        for k in range(NV):
            red[pl.ds(r * D + k * 16, 16)] = (
                red[pl.ds(r * D + k * 16, 16)] + tmp2[r, pl.ds(k * 16, 16)])

    pltpu.sync_copy(red, out_hbm.at[pl.ds(cid * TG * D + rbase, 16 * D)])


def _tc_body(x_ref, z_ref, b_ref, out_ref):
    i = pl.program_id(0)

    @pl.when(i == 0)
    def _():
        out_ref[...] = jnp.zeros((2, G, D), jnp.float32)

    g_min = b_ref[0, 0, 0]
    g_max = b_ref[0, 0, RTC - 1]
    gb = jnp.minimum((g_min // 8) * 8, G - 16)
    fast = g_max < gb + 16

    zrow = z_ref[0]
    brow = b_ref[0]
    vmask = zrow == VIRT
    x = x_ref[...]
    dn = (((1,), (0,)), ((), ()))

    @pl.when(fast)
    def _():
        bb = jnp.broadcast_to(brow, (16, RTC))
        vv = jnp.broadcast_to(vmask, (16, RTC))
        gi = lax.broadcasted_iota(jnp.int32, (16, RTC), 0) + gb
        eq = gi == bb
        sreal = (eq & jnp.logical_not(vv)).astype(jnp.float32)
        svirt = (eq & vv).astype(jnp.float32)
        pr = lax.dot_general(sreal, x, dn, preferred_element_type=jnp.float32)
        pv = lax.dot_general(svirt, x, dn, preferred_element_type=jnp.float32)
        out_ref[0, pl.ds(gb, 16), :] += pr
        out_ref[1, pl.ds(gb, 16), :] += pv

    @pl.when(jnp.logical_not(fast))
    def _():
        bb = jnp.broadcast_to(brow, (G, RTC))
        vv = jnp.broadcast_to(vmask, (G, RTC))
        gi = lax.broadcasted_iota(jnp.int32, (G, RTC), 0)
        eq = gi == bb
        sreal = (eq & jnp.logical_not(vv)).astype(jnp.float32)
        svirt = (eq & vv).astype(jnp.float32)
        pr = lax.dot_general(sreal, x, dn, preferred_element_type=jnp.float32)
        pv = lax.dot_general(svirt, x, dn, preferred_element_type=jnp.float32)
        out_ref[0] += pr
        out_ref[1] += pv


_tc_pool = pl.pallas_call(
    _tc_body,
    grid=(NBT,),
    in_specs=[
        pl.BlockSpec((RTC, D), lambda i: (TC0 + i, 0)),
        pl.BlockSpec((1, 1, RTC), lambda i: (TC0 + i, 0, 0)),
        pl.BlockSpec((1, 1, RTC), lambda i: (TC0 + i, 0, 0)),
    ],
    out_specs=pl.BlockSpec((2, G, D), lambda i: (0, 0, 0)),
    out_shape=jax.ShapeDtypeStruct((2, G, D), jnp.float32),
    compiler_params=pltpu.CompilerParams(
        dimension_semantics=("arbitrary",)),
)


def kernel(out, z_rv, x_rv_batch):
    z32 = z_rv.astype(jnp.int32)
    b32 = x_rv_batch.astype(jnp.int32)
    sc = _pool_kernel(out, z32, b32)
    tc = _tc_pool(out, z32.reshape(N // RTC, 1, RTC),
                  b32.reshape(N // RTC, 1, RTC))
    scp = (sc.reshape(NC, TG * D).sum(0)).reshape(G, 2, D)
    real = scp[:, 0, :] + tc[0]
    virt = scp[:, 1, :] + tc[1]
    return jnp.concatenate((real, virt), axis=1)
